# baseline (speedup 1.0000x reference)
import jax
import jax.numpy as jnp
from jax.experimental import pallas as pl

_BLK = 256


def _cls_proj_kernel(ct_ref, cp_ref, wt_ref, bt_ref, out_ref):
    v = ct_ref[...] + cp_ref[...]
    r = jax.lax.dot_general(
        v, wt_ref[...], (((1,), (1,)), ((), ())),
        preferred_element_type=jnp.float32)
    out_ref[...] = jnp.broadcast_to(r + bt_ref[...], out_ref.shape)


def kernel(pts, colors, W1, b1, g1, bb1, W2, b2, W3, b3, g2, bb2, W4, b4,
           We, be, Wt, bt, cls_token, cls_pos, Wp1, bp1, Wp2, bp2):
    B = pts.shape[0]
    D_in = Wt.shape[1]
    D_out = Wt.shape[0]
    grid = D_out // _BLK
    return pl.pallas_call(
        _cls_proj_kernel,
        grid=(grid,),
        in_specs=[
            pl.BlockSpec((1, D_in), lambda j: (0, 0)),
            pl.BlockSpec((1, D_in), lambda j: (0, 0)),
            pl.BlockSpec((_BLK, D_in), lambda j: (j, 0)),
            pl.BlockSpec((1, _BLK), lambda j: (0, j)),
        ],
        out_specs=pl.BlockSpec((B, _BLK), lambda j: (0, j)),
        out_shape=jax.ShapeDtypeStruct((B, D_out), jnp.float32),
    )(cls_token.reshape(1, D_in), cls_pos.reshape(1, D_in), Wt,
      bt.reshape(1, D_out))


# speedup vs baseline: 1.1904x; 1.1904x over previous
import jax
import jax.numpy as jnp
from jax.experimental import pallas as pl

_BLK = 512


def _cls_proj_kernel(ct_ref, cp_ref, wt_ref, bt_ref, out_ref):
    v = ct_ref[...] + cp_ref[...]
    r = jax.lax.dot_general(
        v, wt_ref[...], (((1,), (1,)), ((), ())),
        preferred_element_type=jnp.float32)
    out_ref[...] = jnp.broadcast_to(r + bt_ref[...], out_ref.shape)


def kernel(pts, colors, W1, b1, g1, bb1, W2, b2, W3, b3, g2, bb2, W4, b4,
           We, be, Wt, bt, cls_token, cls_pos, Wp1, bp1, Wp2, bp2):
    B = pts.shape[0]
    D_in = Wt.shape[1]
    D_out = Wt.shape[0]
    grid = D_out // _BLK
    return pl.pallas_call(
        _cls_proj_kernel,
        grid=(grid,),
        in_specs=[
            pl.BlockSpec((1, D_in), lambda j: (0, 0)),
            pl.BlockSpec((1, D_in), lambda j: (0, 0)),
            pl.BlockSpec((_BLK, D_in), lambda j: (j, 0)),
            pl.BlockSpec((1, _BLK), lambda j: (0, j)),
        ],
        out_specs=pl.BlockSpec((B, _BLK), lambda j: (0, j)),
        out_shape=jax.ShapeDtypeStruct((B, D_out), jnp.float32),
    )(cls_token.reshape(1, D_in), cls_pos.reshape(1, D_in), Wt,
      bt.reshape(1, D_out))
